# Initial kernel scaffold; baseline (speedup 1.0000x reference)
import jax
import jax.numpy as jnp
from jax.experimental import pallas as pl


def kernel(x, edge_index, W1, b1, W2, b2):
    raise NotImplementedError("write your pallas kernel here")


# speedup vs baseline: 13.8420x; 13.8420x over previous
import functools

import jax
import jax.numpy as jnp
from jax import lax
from jax.experimental import pallas as pl
from jax.experimental.pallas import tpu as pltpu
from jax.experimental.pallas import tpu_sc as plsc

N = 10000
E = 320000
D = 128
NP = 10240
NC = 2
NS = 16
L = 16
NW = NC * NS
EW = E // NW
K = 80
CH = EW // K
RPT = NP // NS

_f32 = jnp.float32

_sc_mesh = plsc.VectorSubcoreMesh(
    core_axis_name="c", subcore_axis_name="s", num_cores=NC, num_subcores=NS
)


def _deg_body(dst_hbm, out_hbm, dbuf, hist):
    c = lax.axis_index("c")
    s = lax.axis_index("s")
    wid = s * NC + c

    def zero(i, _):
        hist[pl.ds(pl.multiple_of(i * L, L), L)] = jnp.zeros((L,), _f32)
        return _

    lax.fori_loop(0, NP // L, zero, None)

    pltpu.sync_copy(dst_hbm.at[pl.ds(pl.multiple_of(wid * EW, 8), EW)], dbuf)

    ones = jnp.ones((L,), _f32)

    def body(i, _):
        idx = dbuf[pl.ds(pl.multiple_of(i * L, L), L)]
        plsc.addupdate_scatter(hist, [idx], ones)
        return _

    lax.fori_loop(0, EW // L, body, None)
    pltpu.sync_copy(hist, out_hbm.at[wid])


_deg = functools.partial(
    pl.kernel,
    out_type=jax.ShapeDtypeStruct((NW, NP), _f32),
    mesh=_sc_mesh,
    compiler_params=pltpu.CompilerParams(needs_layout_passes=False),
    scratch_types=[
        pltpu.VMEM((EW,), jnp.int32),
        pltpu.VMEM((NP,), _f32),
    ],
)(_deg_body)


def _agg_body(g_hbm, src_hbm, dst_hbm, out_hbm, sbuf, dbuf, rows, acc, gsem):
    c = lax.axis_index("c")
    s = lax.axis_index("s")
    wid = s * NC + c

    def zrow(r, _):
        for j in range(D // L):
            rows[r, pl.ds(j * L, L)] = jnp.zeros((L,), _f32)
        return _

    lax.fori_loop(0, K, zrow, None)
    base = s * RPT
    for t in range(RPT // K):
        pltpu.sync_copy(rows, acc.at[pl.ds(base + t * K, K)])
    plsc.subcore_barrier()

    eb = wid * EW

    def body(j, _):
        off = pl.multiple_of(eb + j * K, 8)
        pltpu.sync_copy(src_hbm.at[pl.ds(off, K)], sbuf)
        pltpu.sync_copy(dst_hbm.at[pl.ds(off, K)], dbuf)
        pltpu.async_copy(g_hbm.at[sbuf], rows, gsem).wait()
        pltpu.sync_copy(rows, acc.at[dbuf], add=True)
        return _

    lax.fori_loop(0, CH, body, None)
    plsc.subcore_barrier()

    pltpu.sync_copy(
        acc.at[pl.ds(s * RPT, RPT)], out_hbm.at[c, pl.ds(s * RPT, RPT)]
    )


_agg = functools.partial(
    pl.kernel,
    out_type=jax.ShapeDtypeStruct((NC, NP, D), _f32),
    mesh=_sc_mesh,
    compiler_params=pltpu.CompilerParams(needs_layout_passes=False),
    scratch_types=[
        pltpu.VMEM((K,), jnp.int32),
        pltpu.VMEM((K,), jnp.int32),
        pltpu.VMEM((K, D), _f32),
        pltpu.VMEM_SHARED((NP, D), _f32),
        pltpu.SemaphoreType.DMA,
    ],
)(_agg_body)


BN = 512
GRID = NP // BN


def _dinv_of(cnt):
    return lax.rsqrt(jnp.sum(cnt, axis=0) + 1.0)


def _prep_body(x_ref, cnt_ref, w_ref, g_ref):
    dinv = _dinv_of(cnt_ref[...])
    h = jnp.dot(x_ref[...], w_ref[...], preferred_element_type=_f32)
    g_ref[...] = h * dinv[:, None]


def _mid_body(p0_ref, p1_ref, g_ref, cnt_ref, b_ref, w_ref, o_ref):
    dinv = _dinv_of(cnt_ref[...])
    g = g_ref[...]
    s = p0_ref[...] + p1_ref[...] + g
    x1 = s * dinv[:, None] + b_ref[...]
    h2 = jnp.dot(x1, w_ref[...], preferred_element_type=_f32)
    o_ref[...] = h2 * dinv[:, None]


def _fin_body(p0_ref, p1_ref, g_ref, cnt_ref, b_ref, o_ref):
    dinv = _dinv_of(cnt_ref[...])
    g = g_ref[...]
    s = p0_ref[...] + p1_ref[...] + g
    o_ref[...] = jnp.tanh(s * dinv[:, None] + b_ref[...])


_row_spec = pl.BlockSpec((BN, D), lambda i: (i, 0))
_cnt_spec = pl.BlockSpec((NW, BN), lambda i: (0, i))
_w_spec = pl.BlockSpec((D, D), lambda i: (0, 0))
_b_spec = pl.BlockSpec((1, D), lambda i: (0, 0))

_prep = pl.pallas_call(
    _prep_body,
    grid=(GRID,),
    in_specs=[_row_spec, _cnt_spec, _w_spec],
    out_specs=_row_spec,
    out_shape=jax.ShapeDtypeStruct((NP, D), _f32),
)

_mid = pl.pallas_call(
    _mid_body,
    grid=(GRID,),
    in_specs=[_row_spec, _row_spec, _row_spec, _cnt_spec, _b_spec, _w_spec],
    out_specs=_row_spec,
    out_shape=jax.ShapeDtypeStruct((NP, D), _f32),
)

_fin = pl.pallas_call(
    _fin_body,
    grid=(GRID,),
    in_specs=[_row_spec, _row_spec, _row_spec, _cnt_spec, _b_spec],
    out_specs=_row_spec,
    out_shape=jax.ShapeDtypeStruct((NP, D), _f32),
)


@jax.jit
def _run(x, edge_index, W1, b1, W2, b2):
    src = edge_index[0]
    dst = edge_index[1]
    xp = jnp.pad(x, ((0, NP - N), (0, 0)))
    b1r = b1.reshape(1, D)
    b2r = b2.reshape(1, D)

    cnt = _deg(dst)
    g1 = _prep(xp, cnt, W1)
    p = _agg(g1, src, dst)
    g2 = _mid(p[0], p[1], g1, cnt, b1r, W2)
    q = _agg(g2, src, dst)
    out = _fin(q[0], q[1], g2, cnt, b2r)
    return out[:N]


def kernel(x, edge_index, W1, b1, W2, b2):
    return _run(x, edge_index, W1, b1, W2, b2)
